# baseline (speedup 1.0000x reference)
import jax
import jax.numpy as jnp
from jax.experimental import pallas as pl

D = 32
K = 8
ND = D * K
W = 2 * ND + K

BM = 1024


def _body(x_ref, mean_ref, std_ref, pi_ref):
    x = x_ref[...]
    mean_ref[...] = x[:, :ND]
    std_ref[...] = jax.nn.sigmoid(x[:, ND:2 * ND])
    logits = x[:, 2 * ND:]
    m = jnp.max(logits, axis=-1, keepdims=True)
    e = jnp.exp(logits - m)
    pi_ref[...] = e / jnp.sum(e, axis=-1, keepdims=True)


def kernel(x):
    n = x.shape[0]
    grid = (n // BM,)
    mean2d, std2d, pi = pl.pallas_call(
        _body,
        grid=grid,
        in_specs=[pl.BlockSpec((BM, W), lambda i: (i, 0))],
        out_specs=[
            pl.BlockSpec((BM, ND), lambda i: (i, 0)),
            pl.BlockSpec((BM, ND), lambda i: (i, 0)),
            pl.BlockSpec((BM, K), lambda i: (i, 0)),
        ],
        out_shape=[
            jax.ShapeDtypeStruct((n, ND), jnp.float32),
            jax.ShapeDtypeStruct((n, ND), jnp.float32),
            jax.ShapeDtypeStruct((n, K), jnp.float32),
        ],
    )(x)
    return (mean2d.reshape(n, D, K), std2d.reshape(n, D, K), pi)
